# Initial kernel scaffold; baseline (speedup 1.0000x reference)
import jax
import jax.numpy as jnp
from jax.experimental import pallas as pl


def kernel(x, edge_index, W_neigh, W_self, bias):
    raise NotImplementedError("write your pallas kernel here")


# speedup vs baseline: 4.0395x; 4.0395x over previous
import functools

import jax
import jax.numpy as jnp
from jax import lax
from jax.experimental import pallas as pl
from jax.experimental.pallas import tpu as pltpu
from jax.experimental.pallas import tpu_sc as plsc

NC = 2
NS = 16
NW = NC * NS
CH = 128


def _sc_mesh():
    return plsc.VectorSubcoreMesh(core_axis_name="c", subcore_axis_name="s")


def _zero_vmem_f32(ref, rows):
    cols = ref.shape[1]
    per_row = cols // 16
    z = jnp.zeros((16,), jnp.float32)

    def body(i, _):
        ref[i // per_row, pl.ds((i % per_row) * 16, 16)] = z
        return 0

    lax.fori_loop(0, rows * per_row, body, 0)


def _make_sc_agg(n_acc, k, d):
    mesh = _sc_mesh()
    stripe = n_acc // NS

    @functools.partial(
        pl.kernel,
        mesh=mesh,
        out_type=jax.ShapeDtypeStruct((NC, n_acc, d), jnp.float32),
        scratch_types=[
            pltpu.VMEM((k, CH), jnp.int32),
            pltpu.VMEM((k, CH), jnp.int32),
            pltpu.VMEM((CH, d), jnp.float32),
            pltpu.VMEM_SHARED((n_acc, d), jnp.float32),
            pltpu.SemaphoreType.DMA,
        ],
    )
    def agg_kernel(h_hbm, src_hbm, dst_hbm, out_hbm, src_v, dst_v, rows_v, acc, sem):
        cid = lax.axis_index("c")
        sid = lax.axis_index("s")
        wid = cid * NS + sid
        r0 = sid * stripe

        pltpu.sync_copy(src_hbm.at[wid], src_v)
        pltpu.sync_copy(dst_hbm.at[wid], dst_v)
        _zero_vmem_f32(rows_v, CH)
        nfull = stripe // CH
        for t in range(nfull):
            pltpu.sync_copy(rows_v, acc.at[pl.ds(r0 + t * CH, CH)])
        rem = stripe - nfull * CH
        if rem:
            pltpu.sync_copy(rows_v.at[pl.ds(0, rem)], acc.at[pl.ds(r0 + nfull * CH, rem)])
        plsc.subcore_barrier()

        def step(j, _):
            pltpu.async_copy(h_hbm.at[src_v.at[j]], rows_v, sem).wait()
            pltpu.sync_copy(rows_v, acc.at[dst_v.at[j]], add=True)
            return 0

        lax.fori_loop(0, k, step, 0)
        plsc.subcore_barrier()
        pltpu.sync_copy(acc.at[pl.ds(r0, stripe)], out_hbm.at[cid].at[pl.ds(r0, stripe)])

    return agg_kernel


def _make_sc_deg(n_acc, k, d):
    mesh = _sc_mesh()
    stripe = n_acc // NS

    @functools.partial(
        pl.kernel,
        mesh=mesh,
        out_type=jax.ShapeDtypeStruct((NC, n_acc, d), jnp.float32),
        scratch_types=[
            pltpu.VMEM((k, CH), jnp.int32),
            pltpu.VMEM((CH, d), jnp.float32),
            pltpu.VMEM_SHARED((n_acc, d), jnp.float32),
        ],
    )
    def deg_kernel(dst_hbm, out_hbm, dst_v, ones_v, sdeg):
        cid = lax.axis_index("c")
        sid = lax.axis_index("s")
        wid = cid * NS + sid
        r0 = sid * stripe

        pltpu.sync_copy(dst_hbm.at[wid], dst_v)
        _zero_vmem_f32(ones_v, CH)
        nfull = stripe // CH
        for t in range(nfull):
            pltpu.sync_copy(ones_v, sdeg.at[pl.ds(r0 + t * CH, CH)])
        rem = stripe - nfull * CH
        if rem:
            pltpu.sync_copy(ones_v.at[pl.ds(0, rem)], sdeg.at[pl.ds(r0 + nfull * CH, rem)])
        one = jnp.ones((16,), jnp.float32)
        per_row = d // 16

        def fill(i, _):
            ones_v[i // per_row, pl.ds((i % per_row) * 16, 16)] = one
            return 0

        lax.fori_loop(0, CH * per_row, fill, 0)
        plsc.subcore_barrier()

        def step(j, _):
            pltpu.sync_copy(ones_v, sdeg.at[dst_v.at[j]], add=True)
            return 0

        lax.fori_loop(0, k, step, 0)
        plsc.subcore_barrier()
        pltpu.sync_copy(sdeg.at[pl.ds(r0, stripe)], out_hbm.at[cid].at[pl.ds(r0, stripe)])

    return deg_kernel


def _dense_body(h_ref, agg_ref, dg_ref, ws_ref, wn_ref, b_ref, o_ref):
    a = agg_ref[0] + agg_ref[1]
    deg = dg_ref[0] + dg_ref[1]
    inv = 1.0 / jnp.maximum(deg, 1.0)
    hn = a * inv
    o = (
        jnp.dot(h_ref[...], ws_ref[...], preferred_element_type=jnp.float32)
        + jnp.dot(hn, wn_ref[...], preferred_element_type=jnp.float32)
        + b_ref[...]
    )
    o_ref[...] = jnp.maximum(o, 0.0)


def _make_tc_dense(n, n_acc, d, block_rows):
    grid = n // block_rows
    return pl.pallas_call(
        _dense_body,
        grid=(grid,),
        in_specs=[
            pl.BlockSpec((block_rows, d), lambda i: (i, 0)),
            pl.BlockSpec((NC, block_rows, d), lambda i: (0, i, 0)),
            pl.BlockSpec((NC, block_rows, d), lambda i: (0, i, 0)),
            pl.BlockSpec((d, d), lambda i: (0, 0)),
            pl.BlockSpec((d, d), lambda i: (0, 0)),
            pl.BlockSpec((1, d), lambda i: (0, 0)),
        ],
        out_specs=pl.BlockSpec((block_rows, d), lambda i: (i, 0)),
        out_shape=jax.ShapeDtypeStruct((n, d), jnp.float32),
    )


def kernel(x, edge_index, W_neigh, W_self, bias):
    n, d = x.shape
    e = edge_index.shape[1]
    num_layers = W_neigh.shape[0]

    k = -(-e // (NW * CH))
    e_pad = NW * k * CH
    n_acc = -(-(n + 1) // (NS * 8)) * (NS * 8)

    src = edge_index[0]
    dst = edge_index[1]
    pad = e_pad - e
    if pad:
        src = jnp.concatenate([src, jnp.zeros((pad,), jnp.int32)])
        dst = jnp.concatenate([dst, jnp.full((pad,), n, jnp.int32)])
    src3 = src.reshape(NW, k, CH)
    dst3 = dst.reshape(NW, k, CH)

    sc_agg = _make_sc_agg(n_acc, k, d)
    sc_deg = _make_sc_deg(n_acc, k, d)
    tc_dense = _make_tc_dense(n, n_acc, d, block_rows=1000)

    degw = sc_deg(dst3)
    bias2 = bias.reshape(num_layers, 1, d)

    h = x
    for i in range(num_layers):
        agg2 = sc_agg(h, src3, dst3)
        h = tc_dense(h, agg2, degw, W_self[i], W_neigh[i], bias2[i])
    return h
